# baseline (device time: 47139 ns/iter reference)
import jax
import jax.numpy as jnp
from jax import lax
from jax.experimental import pallas as pl
from jax.experimental.pallas import tpu as pltpu

M_OUT = 512
F = 4096
P = 128
C = 8
FC = F // C
K = 1024
FWD_X = tuple(range(C // 2))
FWD_Z = tuple(range(C // 2, C))

_MESH = pl.DeviceIdType.MESH


def kernel(x, dy):
    def body(
        x_ref, dy_ref, out_ref,
        dyv, bsend, brecv, apiece, mypiece, ag,
        dyin_sem,
        ysend_sem, yrecv_sem,
        xsend_sem, xrecv_sem,
        z1send_sem, z1recv_sem,
        xfsend_sem, xfrecv_sem,
        zfsend_sem, zfrecv_sem,
    ):
        mx = lax.axis_index("x")
        my = lax.axis_index("y")
        mz = lax.axis_index("z")
        p = mx * 2 + mz
        q = (1 - mx) * 2 + mz
        r = mx * 2 + (1 - mz)
        s = (1 - mx) * 2 + (1 - mz)
        ypart = (mx, 1 - my, mz)
        xnbr = (1 - mx, my, mz)
        znbr = (mx, my, 1 - mz)

        def exchange(src, dst, send_sem, recv_sem, dev):
            return pltpu.make_async_remote_copy(
                src_ref=src, dst_ref=dst, send_sem=send_sem,
                recv_sem=recv_sem, device_id=dev, device_id_type=_MESH)

        def dy_dma(c):
            return pltpu.make_async_copy(
                dy_ref.at[:, pl.ds(c * FC, FC)], dyv.at[c % 2],
                dyin_sem.at[c % 2])
        dy_dma(0).start()
        dy_dma(1).start()

        xt = jnp.concatenate(
            [
                x_ref[:, pl.ds((1 - my) * M_OUT + p * P, P)].T,
                x_ref[:, pl.ds(my * M_OUT + p * P, P)].T,
            ],
            axis=0,
        )

        for c in range(C):
            cs = pl.ds(c * FC, FC)
            dy_dma(c).wait()
            b_c = lax.dot_general(
                xt, dyv[c % 2],
                dimension_numbers=(((1,), (0,)), ((), ())),
                preferred_element_type=jnp.float32,
            )
            if c + 2 < C:
                dy_dma(c + 2).start()
            bsend[:, cs] = b_c[0:P, :].astype(jnp.bfloat16)
            apiece[:, cs] = b_c[P:2 * P, :]

        barrier_sem = pltpu.get_barrier_semaphore()
        for nbr in (ypart, xnbr, znbr):
            pl.semaphore_signal(barrier_sem, inc=1, device_id=nbr,
                                device_id_type=_MESH)
        pl.semaphore_wait(barrier_sem, 3)

        y_rdmas = []
        for c in range(C):
            cs = pl.ds(c * FC, FC)
            rd = exchange(bsend.at[:, cs], brecv.at[:, cs],
                          ysend_sem.at[c], yrecv_sem.at[c], ypart)
            rd.start()
            y_rdmas.append(rd)

        x_rdmas, z1_rdmas = [], []
        for c in range(C):
            cs = pl.ds(c * FC, FC)
            y_rdmas[c].wait_recv()
            sum_c = apiece[:, cs] + brecv[:, cs].astype(jnp.float32)
            out_ref[pl.ds(p * P, P), cs] = sum_c
            mypiece[:, cs] = sum_c.astype(jnp.bfloat16)
            rd_x = exchange(mypiece.at[:, cs], ag.at[p, :, cs],
                            xsend_sem.at[c], xrecv_sem.at[c], xnbr)
            rd_x.start()
            x_rdmas.append(rd_x)
            rd_z1 = exchange(mypiece.at[:, cs], ag.at[p, :, cs],
                             z1send_sem.at[c], z1recv_sem.at[c], znbr)
            rd_z1.start()
            z1_rdmas.append(rd_z1)

        fwd_rdmas = []
        for c in FWD_X:
            cs = pl.ds(c * FC, FC)
            exchange(mypiece.at[:, cs], ag.at[r, :, cs],
                     z1send_sem.at[c], z1recv_sem.at[c], znbr).wait_recv()
            rd = exchange(ag.at[r, :, cs], ag.at[r, :, cs],
                          xfsend_sem.at[c], xfrecv_sem.at[c], xnbr)
            rd.start()
            fwd_rdmas.append(rd)
            out_ref[pl.ds(r * P, P), cs] = ag[r, :, cs].astype(jnp.float32)
        for c in FWD_Z:
            cs = pl.ds(c * FC, FC)
            exchange(mypiece.at[:, cs], ag.at[q, :, cs],
                     xsend_sem.at[c], xrecv_sem.at[c], xnbr).wait_recv()
            rd = exchange(ag.at[q, :, cs], ag.at[q, :, cs],
                          zfsend_sem.at[c], zfrecv_sem.at[c], znbr)
            rd.start()
            fwd_rdmas.append(rd)
            out_ref[pl.ds(q * P, P), cs] = ag[q, :, cs].astype(jnp.float32)

        for c in FWD_X:
            cs = pl.ds(c * FC, FC)
            exchange(mypiece.at[:, cs], ag.at[q, :, cs],
                     xsend_sem.at[c], xrecv_sem.at[c], xnbr).wait_recv()
            out_ref[pl.ds(q * P, P), cs] = ag[q, :, cs].astype(jnp.float32)
            exchange(mypiece.at[:, cs], ag.at[s, :, cs],
                     xfsend_sem.at[c], xfrecv_sem.at[c], xnbr).wait_recv()
            out_ref[pl.ds(s * P, P), cs] = ag[s, :, cs].astype(jnp.float32)
        for c in FWD_Z:
            cs = pl.ds(c * FC, FC)
            exchange(mypiece.at[:, cs], ag.at[r, :, cs],
                     z1send_sem.at[c], z1recv_sem.at[c], znbr).wait_recv()
            out_ref[pl.ds(r * P, P), cs] = ag[r, :, cs].astype(jnp.float32)
            exchange(mypiece.at[:, cs], ag.at[s, :, cs],
                     zfsend_sem.at[c], zfrecv_sem.at[c], znbr).wait_recv()
            out_ref[pl.ds(s * P, P), cs] = ag[s, :, cs].astype(jnp.float32)

        for rd in y_rdmas + x_rdmas + z1_rdmas + fwd_rdmas:
            rd.wait_send()

    return pl.pallas_call(
        body,
        out_shape=jax.ShapeDtypeStruct((M_OUT, F), jnp.float32),
        in_specs=[
            pl.BlockSpec(memory_space=pltpu.VMEM),
            pl.BlockSpec(memory_space=pl.ANY),
        ],
        out_specs=pl.BlockSpec(memory_space=pltpu.VMEM),
        scratch_shapes=[
            pltpu.VMEM((2, K, FC), jnp.float32),
            pltpu.VMEM((P, F), jnp.bfloat16),
            pltpu.VMEM((P, F), jnp.bfloat16),
            pltpu.VMEM((P, F), jnp.float32),
            pltpu.VMEM((P, F), jnp.bfloat16),
            pltpu.VMEM((4, P, F), jnp.bfloat16),
            pltpu.SemaphoreType.DMA((2,)),
            pltpu.SemaphoreType.DMA((C,)),
            pltpu.SemaphoreType.DMA((C,)),
            pltpu.SemaphoreType.DMA((C,)),
            pltpu.SemaphoreType.DMA((C,)),
            pltpu.SemaphoreType.DMA((C,)),
            pltpu.SemaphoreType.DMA((C,)),
            pltpu.SemaphoreType.DMA((C,)),
            pltpu.SemaphoreType.DMA((C,)),
            pltpu.SemaphoreType.DMA((C,)),
            pltpu.SemaphoreType.DMA((C,)),
        ],
        compiler_params=pltpu.CompilerParams(
            collective_id=0, vmem_limit_bytes=60 * 1024 * 1024
        ),
    )(x, dy)
